# Initial kernel scaffold; baseline (speedup 1.0000x reference)
import jax
import jax.numpy as jnp
from jax.experimental import pallas as pl


def kernel(x, edge_index0, edge_index1, rows0, cols0, vals0, rows1, cols1, vals1, W0, b0, W1, b1, Wlin, blin):
    raise NotImplementedError("write your pallas kernel here")


# speedup vs baseline: 22.5897x; 22.5897x over previous
import functools

import jax
import jax.numpy as jnp
from jax import lax
from jax.experimental import pallas as pl
from jax.experimental.pallas import tpu as pltpu
from jax.experimental.pallas import tpu_sc as plsc

_B = 16
_N0, _N1, _N2 = 10000, 2500, 625
_N0P, _N1P, _N2P = 10240, 2560, 640
_K = 6
_CIN, _C0, _C1 = 3, 64, 128
_LAT = 64
_E0 = 60000
_E1 = 15000
_E1P = 15008
_F0 = _B * _CIN
_F1 = _B * _C0
_F2 = _B * _C1

_mesh = plsc.VectorSubcoreMesh(core_axis_name="c", subcore_axis_name="s")
_sc_params = pltpu.CompilerParams(needs_layout_passes=False)


def _wid():
    return lax.axis_index("s") * 2 + lax.axis_index("c")


def _zeros16():
    return jnp.zeros((16,), jnp.float32)


def _rsqrt16(d):
    x = plsc.bitcast(d, jnp.int32)
    x = jnp.full((16,), 0x5F3759DF, jnp.int32) - lax.shift_right_logical(x, 1)
    y = plsc.bitcast(x, jnp.float32)
    for _ in range(3):
        y = y * (1.5 - (0.5 * d) * (y * y))
    return y


def _chunks(total, step):
    out = []
    off = 0
    while off < total:
        out.append((off, min(step, total - off)))
        off += step
    return out


@functools.partial(
    pl.kernel, mesh=_mesh, compiler_params=_sc_params,
    out_type=[jax.ShapeDtypeStruct((_E0,), jnp.float32),
              jax.ShapeDtypeStruct((_E1P,), jnp.float32)],
    scratch_types=[pltpu.VMEM((_N0P,), jnp.float32),
                   pltpu.VMEM((2048,), jnp.int32),
                   pltpu.VMEM((2048,), jnp.int32),
                   pltpu.VMEM((2048,), jnp.float32)],
)
def _norm_kernel(r0_hbm, c0_hbm, r1_hbm, c1_hbm, n0_hbm, n1_hbm,
                 dis, rb, cb, ob):
    wid = _wid()
    ones = jnp.full((16,), 1.0, jnp.float32)

    def level(rows_hbm, cols_hbm, out_hbm, n_nodes, n_pad, n_edges):
        def z(i, c):
            dis[pl.ds(i * 16, 16)] = _zeros16()
            return c
        lax.fori_loop(0, n_pad // 16, z, 0)
        for off, sz in _chunks(n_edges, 2048):
            pltpu.sync_copy(rows_hbm.at[pl.ds(off, sz)], rb.at[pl.ds(0, sz)])

            def body(v, c):
                r16 = rb[pl.ds(v * 16, 16)]
                plsc.addupdate_scatter(dis, [r16], ones)
                return c
            lax.fori_loop(0, sz // 16, body, 0)
        def dz(i, c):
            d = dis[pl.ds(i * 16, 16)]
            y = _rsqrt16(jnp.maximum(d, 1.0))
            dis[pl.ds(i * 16, 16)] = jnp.where(d > 0.0, y, 0.0)
            return c
        lax.fori_loop(0, n_pad // 16, dz, 0)
        for off, sz in _chunks(n_edges, 2048):
            pltpu.sync_copy(rows_hbm.at[pl.ds(off, sz)], rb.at[pl.ds(0, sz)])
            pltpu.sync_copy(cols_hbm.at[pl.ds(off, sz)], cb.at[pl.ds(0, sz)])

            def body(v, c):
                r16 = rb[pl.ds(v * 16, 16)]
                c16 = cb[pl.ds(v * 16, 16)]
                dr = plsc.load_gather(dis, [r16])
                dc = plsc.load_gather(dis, [c16])
                n = -(dr * dc)
                n = jnp.where(r16 < n_nodes, n, 0.0)
                ob[pl.ds(v * 16, 16)] = n
                return c
            lax.fori_loop(0, sz // 16, body, 0)
            pltpu.sync_copy(ob.at[pl.ds(0, sz)], out_hbm.at[pl.ds(off, sz)])

    @pl.when(wid == 0)
    def _():
        level(r0_hbm, c0_hbm, n0_hbm, _N0, _N0P, _E0)

    @pl.when(wid == 1)
    def _():
        level(r1_hbm, c1_hbm, n1_hbm, _N1, _N1P, _E1P)


_E0CH = 4096

@functools.partial(
    pl.kernel, mesh=_mesh, compiler_params=_sc_params,
    out_type=jax.ShapeDtypeStruct((_K - 1, _F0, _N0P), jnp.float32),
    scratch_types=[pltpu.VMEM((_N0P,), jnp.float32),
                   pltpu.VMEM((_N0P,), jnp.float32),
                   pltpu.VMEM((_N0P,), jnp.float32),
                   pltpu.VMEM((_N0P,), jnp.float32),
                   pltpu.VMEM((_E0CH,), jnp.int32),
                   pltpu.VMEM((_E0CH,), jnp.int32),
                   pltpu.VMEM((_E0CH,), jnp.float32),
                   pltpu.SemaphoreType.DMA],
)
def _cheb0_kernel(x_hbm, r_hbm, c_hbm, n_hbm, tcat_hbm,
                  a0, b0, a1, b1, rb, cb, nb, sem):
    wid = _wid()
    ch0 = wid
    ch1 = wid + 32
    has2 = wid < 16

    nvz = _N0P // 16

    pltpu.sync_copy(x_hbm.at[ch0], a0)

    @pl.when(has2)
    def _():
        pltpu.sync_copy(x_hbm.at[ch1], a1)

    def zb(i, c):
        b0[pl.ds(i * 16, 16)] = _zeros16()
        b1[pl.ds(i * 16, 16)] = _zeros16()
        return c
    lax.fori_loop(0, nvz, zb, 0)

    pairs = [(a0, b0), (a1, b1)]
    for k in range(1, _K):
        if k >= 2:
            def neg(i, c, d0=pairs[0][1], d1=pairs[1][1]):
                d0[pl.ds(i * 16, 16)] = -d0[pl.ds(i * 16, 16)]
                d1[pl.ds(i * 16, 16)] = -d1[pl.ds(i * 16, 16)]
                return c
            lax.fori_loop(0, nvz, neg, 0)
        for off, sz in _chunks(_E0, _E0CH):
            h1 = pltpu.async_copy(r_hbm.at[pl.ds(off, sz)], rb.at[pl.ds(0, sz)], sem)
            h2 = pltpu.async_copy(c_hbm.at[pl.ds(off, sz)], cb.at[pl.ds(0, sz)], sem)
            h3 = pltpu.async_copy(n_hbm.at[pl.ds(off, sz)], nb.at[pl.ds(0, sz)], sem)
            h1.wait(); h2.wait(); h3.wait()

            def body(v, c):
                r16 = rb[pl.ds(v * 16, 16)]
                c16 = cb[pl.ds(v * 16, 16)]
                n16 = nb[pl.ds(v * 16, 16)]
                nn = n16 if k == 1 else n16 + n16
                s0, d0 = pairs[0]
                g = plsc.load_gather(s0, [c16])
                plsc.addupdate_scatter(d0, [r16], g * nn)

                @pl.when(has2)
                def _():
                    s1, d1 = pairs[1]
                    g1 = plsc.load_gather(s1, [c16])
                    plsc.addupdate_scatter(d1, [r16], g1 * nn)
                return c
            lax.fori_loop(0, sz // 16, body, 0)
        dst0 = pairs[0][1]
        pltpu.sync_copy(dst0, tcat_hbm.at[k - 1, ch0])

        @pl.when(has2)
        def _():
            dst1 = pairs[1][1]
            pltpu.sync_copy(dst1, tcat_hbm.at[k - 1, ch1])
        pairs = [(pairs[0][1], pairs[0][0]), (pairs[1][1], pairs[1][0])]


@functools.partial(
    pl.kernel, mesh=_mesh, compiler_params=_sc_params,
    out_type=jax.ShapeDtypeStruct((_K - 1, _F1, _N1P), jnp.float32),
    scratch_types=[pltpu.VMEM((16, _N1P), jnp.float32),
                   pltpu.VMEM((16, _N1P), jnp.float32),
                   pltpu.VMEM((_E1P,), jnp.int32),
                   pltpu.VMEM((_E1P,), jnp.int32),
                   pltpu.VMEM((_E1P,), jnp.float32),
                   pltpu.SemaphoreType.DMA],
)
def _cheb1_kernel(x_hbm, r_hbm, c_hbm, n_hbm, tcat_hbm,
                  abuf, bbuf, rb, cb, nb, sem):
    wid = _wid()
    h1 = pltpu.async_copy(r_hbm, rb, sem)
    h2 = pltpu.async_copy(c_hbm, cb, sem)
    h3 = pltpu.async_copy(n_hbm, nb, sem)
    h1.wait(); h2.wait(); h3.wait()

    nvz = _N1P // 16
    nv = _E1P // 16

    for p in range(2):
        ch0 = wid * 32 + p * 16
        pltpu.sync_copy(x_hbm.at[pl.ds(ch0, 16)], abuf)

        def zb(i, c):
            for j in range(16):
                bbuf[j, pl.ds(i * 16, 16)] = _zeros16()
            return c
        lax.fori_loop(0, nvz, zb, 0)

        src, dst = abuf, bbuf
        for k in range(1, _K):
            if k >= 2:
                def neg(i, c):
                    for j in range(16):
                        dst[j, pl.ds(i * 16, 16)] = -dst[j, pl.ds(i * 16, 16)]
                    return c
                lax.fori_loop(0, nvz, neg, 0)

            def body(v, c):
                r16 = rb[pl.ds(v * 16, 16)]
                c16 = cb[pl.ds(v * 16, 16)]
                n16 = nb[pl.ds(v * 16, 16)]
                nn = n16 if k == 1 else n16 + n16
                for j in range(16):
                    jj = jnp.full((16,), j, jnp.int32)
                    g = plsc.load_gather(src, [jj, c16])
                    plsc.addupdate_scatter(dst, [jj, r16], g * nn)
                return c
            lax.fori_loop(0, nv, body, 0)
            pltpu.sync_copy(dst, tcat_hbm.at[k - 1, pl.ds(ch0, 16)])
            src, dst = dst, src


def _make_pool(F, NIN, NOUTP):
    per_tile = F // 32

    @functools.partial(
        pl.kernel, mesh=_mesh, compiler_params=_sc_params,
        out_type=jax.ShapeDtypeStruct((F, NOUTP), jnp.float32),
        scratch_types=[pltpu.VMEM((NIN,), jnp.float32),
                       pltpu.VMEM((4, NOUTP), jnp.int32),
                       pltpu.VMEM((4, NOUTP), jnp.float32),
                       pltpu.VMEM((NOUTP,), jnp.float32)],
    )
    def pool(h_hbm, cols_hbm, vals_hbm, out_hbm, xin, cbuf, vbuf, ob):
        wid = _wid()
        pltpu.sync_copy(cols_hbm, cbuf)
        pltpu.sync_copy(vals_hbm, vbuf)
        for ch in range(per_tile):
            c = wid * per_tile + ch
            pltpu.sync_copy(h_hbm.at[c], xin)

            def body(v, carry):
                acc = _zeros16()
                for j in range(4):
                    idx = cbuf[j, pl.ds(v * 16, 16)]
                    g = plsc.load_gather(xin, [idx])
                    acc = acc + g * vbuf[j, pl.ds(v * 16, 16)]
                ob[pl.ds(v * 16, 16)] = acc
                return carry
            lax.fori_loop(0, NOUTP // 16, body, 0)
            pltpu.sync_copy(ob, out_hbm.at[c])

    return pool


_pool0 = _make_pool(_F1, _N0P, _N1P)
_pool1 = _make_pool(_F2, _N1P, _N2P)


def _elu(x):
    return jnp.where(x > 0.0, x, jnp.exp(jnp.minimum(x, 0.0)) - 1.0)


def _make_mm(CIN, COUT, NP):
    def body(x_ref, t_ref, w_ref, b_ref, o_ref):
        xb = x_ref[...].reshape(CIN, NP)
        tb = t_ref[...].reshape((_K - 1) * CIN, NP)
        lhs = jnp.concatenate([xb, tb], axis=0)
        o = lax.dot_general(w_ref[...], lhs, (((0,), (0,)), ((), ())),
                            preferred_element_type=jnp.float32)
        o_ref[...] = _elu(o + b_ref[...])

    return pl.pallas_call(
        body,
        grid=(_B,),
        in_specs=[
            pl.BlockSpec((1, CIN, NP), lambda b: (b, 0, 0)),
            pl.BlockSpec((_K - 1, 1, CIN, NP), lambda b: (0, b, 0, 0)),
            pl.BlockSpec((_K * CIN, COUT), lambda b: (0, 0)),
            pl.BlockSpec((COUT, 1), lambda b: (0, 0)),
        ],
        out_specs=pl.BlockSpec((COUT, NP), lambda b: (b, 0)),
        out_shape=jax.ShapeDtypeStruct((_B * COUT, NP), jnp.float32),
    )


_mm0 = _make_mm(_CIN, _C0, _N0P)
_mm1 = _make_mm(_C0, _C1, _N1P)


_KF = _C1 * _N2P
_KFB = 8192


def _mmf_body(l_ref, w_ref, b_ref, o_ref):
    i = pl.program_id(0)

    @pl.when(i == 0)
    def _():
        o_ref[...] = jnp.broadcast_to(b_ref[...], (_B, _LAT))

    o_ref[...] += lax.dot_general(
        l_ref[...], w_ref[...], (((1,), (0,)), ((), ())),
        preferred_element_type=jnp.float32)


_mmf = pl.pallas_call(
    _mmf_body,
    grid=(_KF // _KFB,),
    in_specs=[pl.BlockSpec((_B, _KFB), lambda i: (0, i)),
              pl.BlockSpec((_KFB, _LAT), lambda i: (i, 0)),
              pl.BlockSpec((1, _LAT), lambda i: (0, 0))],
    out_specs=pl.BlockSpec((_B, _LAT), lambda i: (0, 0)),
    out_shape=jax.ShapeDtypeStruct((_B, _LAT), jnp.float32),
)


def _deal(arr, lanes=16):
    r = arr.shape[0] // lanes
    return arr.reshape(lanes, r).T.reshape(-1)


def kernel(x, edge_index0, edge_index1, rows0, cols0, vals0, rows1, cols1,
           vals1, W0, b0, W1, b1, Wlin, blin):
    f32, i32 = jnp.float32, jnp.int32

    r0, c0 = edge_index0[0], edge_index0[1]
    p0 = jnp.argsort(r0)
    r0d = _deal(r0[p0]).astype(i32)
    c0d = _deal(c0[p0]).astype(i32)

    pad = _N1 + jnp.arange(_E1P - _E1, dtype=i32)
    r1 = jnp.concatenate([edge_index1[0].astype(i32), pad])
    c1 = jnp.concatenate([edge_index1[1].astype(i32), pad])
    p1 = jnp.argsort(r1)
    r1d = _deal(r1[p1])
    c1d = _deal(c1[p1])

    cols0r = jnp.pad(cols0.reshape(_N1, 4).T, ((0, 0), (0, _N1P - _N1))).astype(i32)
    vals0r = jnp.pad(vals0.reshape(_N1, 4).T, ((0, 0), (0, _N1P - _N1))).astype(f32)
    cols1r = jnp.pad(cols1.reshape(_N2, 4).T, ((0, 0), (0, _N2P - _N2))).astype(i32)
    vals1r = jnp.pad(vals1.reshape(_N2, 4).T, ((0, 0), (0, _N2P - _N2))).astype(f32)

    x0t = jnp.pad(x.transpose(0, 2, 1).reshape(_F0, _N0),
                  ((0, 0), (0, _N0P - _N0)))

    w0r = W0.reshape(_K * _CIN, _C0)
    w1r = W1.reshape(_K * _C0, _C1)
    b0c = b0.reshape(_C0, 1)
    b1c = b1.reshape(_C1, 1)
    wlin_t = jnp.pad(Wlin.reshape(_N2, _C1, _LAT).transpose(1, 0, 2),
                     ((0, 0), (0, _N2P - _N2), (0, 0))).reshape(_C1 * _N2P, _LAT)
    blin2 = blin.reshape(1, _LAT)

    n0, n1 = _norm_kernel(r0d, c0d, r1d, c1d)
    tcat0 = _cheb0_kernel(x0t, r0d, c0d, n0)
    h0 = _mm0(x0t.reshape(_B, _CIN, _N0P),
              tcat0.reshape(_K - 1, _B, _CIN, _N0P), w0r, b0c)
    x1 = _pool0(h0, cols0r, vals0r)
    tcat1 = _cheb1_kernel(x1, r1d, c1d, n1)
    h1 = _mm1(x1.reshape(_B, _C0, _N1P),
              tcat1.reshape(_K - 1, _B, _C0, _N1P), w1r, b1c)
    h1p = _pool1(h1, cols1r, vals1r)
    lhs = h1p.reshape(_B, _C1 * _N2P)
    return _mmf(lhs, wlin_t, blin2)


# speedup vs baseline: 39.9667x; 1.7692x over previous
import functools

import jax
import jax.numpy as jnp
from jax import lax
from jax.experimental import pallas as pl
from jax.experimental.pallas import tpu as pltpu
from jax.experimental.pallas import tpu_sc as plsc

_B = 16
_N0, _N1, _N2 = 10000, 2500, 625
_N0P, _N1P, _N2P = 10240, 2560, 640
_K = 6
_CIN, _C0, _C1 = 3, 64, 128
_LAT = 64
_E0 = 60000
_E1 = 15000
_E1P = 15008
_F0 = _B * _CIN
_F1 = _B * _C0
_F2 = _B * _C1

_mesh = plsc.VectorSubcoreMesh(core_axis_name="c", subcore_axis_name="s")
_sc_params = pltpu.CompilerParams(needs_layout_passes=False)


def _wid():
    return lax.axis_index("s") * 2 + lax.axis_index("c")


def _zeros16():
    return jnp.zeros((16,), jnp.float32)


def _rsqrt16(d):
    x = plsc.bitcast(d, jnp.int32)
    x = jnp.full((16,), 0x5F3759DF, jnp.int32) - lax.shift_right_logical(x, 1)
    y = plsc.bitcast(x, jnp.float32)
    for _ in range(3):
        y = y * (1.5 - (0.5 * d) * (y * y))
    return y


def _chunks(total, step):
    out = []
    off = 0
    while off < total:
        out.append((off, min(step, total - off)))
        off += step
    return out


@functools.partial(
    pl.kernel, mesh=_mesh, compiler_params=_sc_params,
    out_type=[jax.ShapeDtypeStruct((_E0,), jnp.float32),
              jax.ShapeDtypeStruct((_E1P,), jnp.float32)],
    scratch_types=[pltpu.VMEM((_N0P,), jnp.float32),
                   pltpu.VMEM((2048,), jnp.int32),
                   pltpu.VMEM((2048,), jnp.int32),
                   pltpu.VMEM((2048,), jnp.float32)],
)
def _norm_kernel(r0_hbm, c0_hbm, r1_hbm, c1_hbm, n0_hbm, n1_hbm,
                 dis, rb, cb, ob):
    wid = _wid()
    ones = jnp.full((16,), 1.0, jnp.float32)

    def level(rows_hbm, cols_hbm, out_hbm, n_nodes, n_pad, n_edges):
        def z(i, c):
            dis[pl.ds(i * 16, 16)] = _zeros16()
            return c
        lax.fori_loop(0, n_pad // 16, z, 0)
        for off, sz in _chunks(n_edges, 2048):
            pltpu.sync_copy(rows_hbm.at[pl.ds(off, sz)], rb.at[pl.ds(0, sz)])

            def body(v, c):
                r16 = rb[pl.ds(v * 16, 16)]
                plsc.addupdate_scatter(dis, [r16], ones)
                return c
            lax.fori_loop(0, sz // 16, body, 0)
        def dz(i, c):
            d = dis[pl.ds(i * 16, 16)]
            y = _rsqrt16(jnp.maximum(d, 1.0))
            dis[pl.ds(i * 16, 16)] = jnp.where(d > 0.0, y, 0.0)
            return c
        lax.fori_loop(0, n_pad // 16, dz, 0)
        for off, sz in _chunks(n_edges, 2048):
            pltpu.sync_copy(rows_hbm.at[pl.ds(off, sz)], rb.at[pl.ds(0, sz)])
            pltpu.sync_copy(cols_hbm.at[pl.ds(off, sz)], cb.at[pl.ds(0, sz)])

            def body(v, c):
                r16 = rb[pl.ds(v * 16, 16)]
                c16 = cb[pl.ds(v * 16, 16)]
                dr = plsc.load_gather(dis, [r16])
                dc = plsc.load_gather(dis, [c16])
                n = -(dr * dc)
                n = jnp.where(r16 < n_nodes, n, 0.0)
                ob[pl.ds(v * 16, 16)] = n
                return c
            lax.fori_loop(0, sz // 16, body, 0)
            pltpu.sync_copy(ob.at[pl.ds(0, sz)], out_hbm.at[pl.ds(off, sz)])

    @pl.when(wid == 0)
    def _():
        level(r0_hbm, c0_hbm, n0_hbm, _N0, _N0P, _E0)

    @pl.when(wid == 1)
    def _():
        level(r1_hbm, c1_hbm, n1_hbm, _N1, _N1P, _E1P)


_E0CH = 4096

@functools.partial(
    pl.kernel, mesh=_mesh, compiler_params=_sc_params,
    out_type=jax.ShapeDtypeStruct((_K - 1, _F0, _N0P), jnp.float32),
    scratch_types=[pltpu.VMEM((_N0P,), jnp.float32),
                   pltpu.VMEM((_N0P,), jnp.float32),
                   pltpu.VMEM((_N0P,), jnp.float32),
                   pltpu.VMEM((_N0P,), jnp.float32),
                   pltpu.VMEM((_E0CH,), jnp.int32),
                   pltpu.VMEM((_E0CH,), jnp.int32),
                   pltpu.VMEM((_E0CH,), jnp.float32),
                   pltpu.SemaphoreType.DMA],
)
def _cheb0_kernel(x_hbm, r_hbm, c_hbm, n_hbm, tcat_hbm,
                  a0, b0, a1, b1, rb, cb, nb, sem):
    wid = _wid()
    ch0 = wid
    ch1 = wid + 32
    has2 = wid < 16

    nvz = _N0P // 16

    pltpu.sync_copy(x_hbm.at[ch0], a0)

    @pl.when(has2)
    def _():
        pltpu.sync_copy(x_hbm.at[ch1], a1)

    def zb(i, c):
        b0[pl.ds(i * 16, 16)] = _zeros16()
        b1[pl.ds(i * 16, 16)] = _zeros16()
        return c
    lax.fori_loop(0, nvz, zb, 0)

    pairs = [(a0, b0), (a1, b1)]
    for k in range(1, _K):
        if k >= 2:
            def neg(i, c, d0=pairs[0][1], d1=pairs[1][1]):
                d0[pl.ds(i * 16, 16)] = -d0[pl.ds(i * 16, 16)]
                d1[pl.ds(i * 16, 16)] = -d1[pl.ds(i * 16, 16)]
                return c
            lax.fori_loop(0, nvz, neg, 0)
        for off, sz in _chunks(_E0, _E0CH):
            h1 = pltpu.async_copy(r_hbm.at[pl.ds(off, sz)], rb.at[pl.ds(0, sz)], sem)
            h2 = pltpu.async_copy(c_hbm.at[pl.ds(off, sz)], cb.at[pl.ds(0, sz)], sem)
            h3 = pltpu.async_copy(n_hbm.at[pl.ds(off, sz)], nb.at[pl.ds(0, sz)], sem)
            h1.wait(); h2.wait(); h3.wait()

            @plsc.parallel_loop(0, sz // 16, 1, unroll=4)
            def _(v):
                r16 = rb[pl.ds(v * 16, 16)]
                c16 = cb[pl.ds(v * 16, 16)]
                n16 = nb[pl.ds(v * 16, 16)]
                nn = n16 if k == 1 else n16 + n16
                s0, d0 = pairs[0]
                g = plsc.load_gather(s0, [c16])
                plsc.addupdate_scatter(d0, [r16], g * nn)

                @pl.when(has2)
                def _():
                    s1, d1 = pairs[1]
                    g1 = plsc.load_gather(s1, [c16])
                    plsc.addupdate_scatter(d1, [r16], g1 * nn)
        dst0 = pairs[0][1]
        pltpu.sync_copy(dst0, tcat_hbm.at[k - 1, ch0])

        @pl.when(has2)
        def _():
            dst1 = pairs[1][1]
            pltpu.sync_copy(dst1, tcat_hbm.at[k - 1, ch1])
        pairs = [(pairs[0][1], pairs[0][0]), (pairs[1][1], pairs[1][0])]


@functools.partial(
    pl.kernel, mesh=_mesh, compiler_params=_sc_params,
    out_type=jax.ShapeDtypeStruct((_K - 1, _F1, _N1P), jnp.float32),
    scratch_types=[pltpu.VMEM((16, _N1P), jnp.float32),
                   pltpu.VMEM((16, _N1P), jnp.float32),
                   pltpu.VMEM((_E1P,), jnp.int32),
                   pltpu.VMEM((_E1P,), jnp.int32),
                   pltpu.VMEM((_E1P,), jnp.float32),
                   pltpu.SemaphoreType.DMA],
)
def _cheb1_kernel(x_hbm, r_hbm, c_hbm, n_hbm, tcat_hbm,
                  abuf, bbuf, rb, cb, nb, sem):
    wid = _wid()
    h1 = pltpu.async_copy(r_hbm, rb, sem)
    h2 = pltpu.async_copy(c_hbm, cb, sem)
    h3 = pltpu.async_copy(n_hbm, nb, sem)
    h1.wait(); h2.wait(); h3.wait()

    nvz = _N1P // 16
    nv = _E1P // 16

    for p in range(2):
        ch0 = wid * 32 + p * 16
        pltpu.sync_copy(x_hbm.at[pl.ds(ch0, 16)], abuf)

        def zb(i, c):
            for j in range(16):
                bbuf[j, pl.ds(i * 16, 16)] = _zeros16()
            return c
        lax.fori_loop(0, nvz, zb, 0)

        src, dst = abuf, bbuf
        for k in range(1, _K):
            if k >= 2:
                def neg(i, c):
                    for j in range(16):
                        dst[j, pl.ds(i * 16, 16)] = -dst[j, pl.ds(i * 16, 16)]
                    return c
                lax.fori_loop(0, nvz, neg, 0)

            @plsc.parallel_loop(0, nv, 1, unroll=2)
            def _(v):
                r16 = rb[pl.ds(v * 16, 16)]
                c16 = cb[pl.ds(v * 16, 16)]
                n16 = nb[pl.ds(v * 16, 16)]
                nn = n16 if k == 1 else n16 + n16
                for j in range(16):
                    jj = jnp.full((16,), j, jnp.int32)
                    g = plsc.load_gather(src, [jj, c16])
                    plsc.addupdate_scatter(dst, [jj, r16], g * nn)
            pltpu.sync_copy(dst, tcat_hbm.at[k - 1, pl.ds(ch0, 16)])
            src, dst = dst, src


def _make_pool(F, NIN, NOUTP):
    per_tile = F // 32

    @functools.partial(
        pl.kernel, mesh=_mesh, compiler_params=_sc_params,
        out_type=jax.ShapeDtypeStruct((F, NOUTP), jnp.float32),
        scratch_types=[pltpu.VMEM((NIN,), jnp.float32),
                       pltpu.VMEM((4, NOUTP), jnp.int32),
                       pltpu.VMEM((4, NOUTP), jnp.float32),
                       pltpu.VMEM((NOUTP,), jnp.float32)],
    )
    def pool(h_hbm, cols_hbm, vals_hbm, out_hbm, xin, cbuf, vbuf, ob):
        wid = _wid()
        pltpu.sync_copy(cols_hbm, cbuf)
        pltpu.sync_copy(vals_hbm, vbuf)
        for ch in range(per_tile):
            c = wid * per_tile + ch
            pltpu.sync_copy(h_hbm.at[c], xin)

            @plsc.parallel_loop(0, NOUTP // 16, 1, unroll=4)
            def _(v):
                acc = _zeros16()
                for j in range(4):
                    idx = cbuf[j, pl.ds(v * 16, 16)]
                    g = plsc.load_gather(xin, [idx])
                    acc = acc + g * vbuf[j, pl.ds(v * 16, 16)]
                ob[pl.ds(v * 16, 16)] = acc
            pltpu.sync_copy(ob, out_hbm.at[c])

    return pool


_pool0 = _make_pool(_F1, _N0P, _N1P)
_pool1 = _make_pool(_F2, _N1P, _N2P)


def _elu(x):
    return jnp.where(x > 0.0, x, jnp.exp(jnp.minimum(x, 0.0)) - 1.0)


def _make_mm(CIN, COUT, NP):
    def body(x_ref, t_ref, w_ref, b_ref, o_ref):
        xb = x_ref[...].reshape(CIN, NP)
        tb = t_ref[...].reshape((_K - 1) * CIN, NP)
        lhs = jnp.concatenate([xb, tb], axis=0)
        o = lax.dot_general(w_ref[...], lhs, (((0,), (0,)), ((), ())),
                            preferred_element_type=jnp.float32)
        o_ref[...] = _elu(o + b_ref[...])

    return pl.pallas_call(
        body,
        grid=(_B,),
        in_specs=[
            pl.BlockSpec((1, CIN, NP), lambda b: (b, 0, 0)),
            pl.BlockSpec((_K - 1, 1, CIN, NP), lambda b: (0, b, 0, 0)),
            pl.BlockSpec((_K * CIN, COUT), lambda b: (0, 0)),
            pl.BlockSpec((COUT, 1), lambda b: (0, 0)),
        ],
        out_specs=pl.BlockSpec((COUT, NP), lambda b: (b, 0)),
        out_shape=jax.ShapeDtypeStruct((_B * COUT, NP), jnp.float32),
    )


_mm0 = _make_mm(_CIN, _C0, _N0P)
_mm1 = _make_mm(_C0, _C1, _N1P)


_KF = _C1 * _N2P
_KFB = 8192


def _mmf_body(l_ref, w_ref, b_ref, o_ref):
    i = pl.program_id(0)

    @pl.when(i == 0)
    def _():
        o_ref[...] = jnp.broadcast_to(b_ref[...], (_B, _LAT))

    o_ref[...] += lax.dot_general(
        l_ref[...], w_ref[...], (((1,), (0,)), ((), ())),
        preferred_element_type=jnp.float32)


_mmf = pl.pallas_call(
    _mmf_body,
    grid=(_KF // _KFB,),
    in_specs=[pl.BlockSpec((_B, _KFB), lambda i: (0, i)),
              pl.BlockSpec((_KFB, _LAT), lambda i: (i, 0)),
              pl.BlockSpec((1, _LAT), lambda i: (0, 0))],
    out_specs=pl.BlockSpec((_B, _LAT), lambda i: (0, 0)),
    out_shape=jax.ShapeDtypeStruct((_B, _LAT), jnp.float32),
)


def _deal(arr, lanes=16):
    r = arr.shape[0] // lanes
    return arr.reshape(lanes, r).T.reshape(-1)


def kernel(x, edge_index0, edge_index1, rows0, cols0, vals0, rows1, cols1,
           vals1, W0, b0, W1, b1, Wlin, blin):
    f32, i32 = jnp.float32, jnp.int32

    r0, c0 = edge_index0[0], edge_index0[1]
    p0 = jnp.argsort(r0)
    r0d = _deal(r0[p0]).astype(i32)
    c0d = _deal(c0[p0]).astype(i32)

    pad = _N1 + jnp.arange(_E1P - _E1, dtype=i32)
    r1 = jnp.concatenate([edge_index1[0].astype(i32), pad])
    c1 = jnp.concatenate([edge_index1[1].astype(i32), pad])
    p1 = jnp.argsort(r1)
    r1d = _deal(r1[p1])
    c1d = _deal(c1[p1])

    cols0r = jnp.pad(cols0.reshape(_N1, 4).T, ((0, 0), (0, _N1P - _N1))).astype(i32)
    vals0r = jnp.pad(vals0.reshape(_N1, 4).T, ((0, 0), (0, _N1P - _N1))).astype(f32)
    cols1r = jnp.pad(cols1.reshape(_N2, 4).T, ((0, 0), (0, _N2P - _N2))).astype(i32)
    vals1r = jnp.pad(vals1.reshape(_N2, 4).T, ((0, 0), (0, _N2P - _N2))).astype(f32)

    x0t = jnp.pad(x.transpose(0, 2, 1).reshape(_F0, _N0),
                  ((0, 0), (0, _N0P - _N0)))

    w0r = W0.reshape(_K * _CIN, _C0)
    w1r = W1.reshape(_K * _C0, _C1)
    b0c = b0.reshape(_C0, 1)
    b1c = b1.reshape(_C1, 1)
    wlin_t = jnp.pad(Wlin.reshape(_N2, _C1, _LAT).transpose(1, 0, 2),
                     ((0, 0), (0, _N2P - _N2), (0, 0))).reshape(_C1 * _N2P, _LAT)
    blin2 = blin.reshape(1, _LAT)

    n0, n1 = _norm_kernel(r0d, c0d, r1d, c1d)
    tcat0 = _cheb0_kernel(x0t, r0d, c0d, n0)
    h0 = _mm0(x0t.reshape(_B, _CIN, _N0P),
              tcat0.reshape(_K - 1, _B, _CIN, _N0P), w0r, b0c)
    x1 = _pool0(h0, cols0r, vals0r)
    tcat1 = _cheb1_kernel(x1, r1d, c1d, n1)
    h1 = _mm1(x1.reshape(_B, _C0, _N1P),
              tcat1.reshape(_K - 1, _B, _C0, _N1P), w1r, b1c)
    h1p = _pool1(h1, cols1r, vals1r)
    lhs = h1p.reshape(_B, _C1 * _N2P)
    return _mmf(lhs, wlin_t, blin2)


# speedup vs baseline: 41.0839x; 1.0280x over previous
import functools

import jax
import jax.numpy as jnp
from jax import lax
from jax.experimental import pallas as pl
from jax.experimental.pallas import tpu as pltpu
from jax.experimental.pallas import tpu_sc as plsc

_B = 16
_N0, _N1, _N2 = 10000, 2500, 625
_N0P, _N1P, _N2P = 10240, 2560, 640
_K = 6
_CIN, _C0, _C1 = 3, 64, 128
_LAT = 64
_E0 = 60000
_E1 = 15000
_E1P = 15008
_F0 = _B * _CIN
_F1 = _B * _C0
_F2 = _B * _C1

_mesh = plsc.VectorSubcoreMesh(core_axis_name="c", subcore_axis_name="s")
_sc_params = pltpu.CompilerParams(needs_layout_passes=False)


def _wid():
    return lax.axis_index("s") * 2 + lax.axis_index("c")


def _zeros16():
    return jnp.zeros((16,), jnp.float32)


def _rsqrt16(d):
    x = plsc.bitcast(d, jnp.int32)
    x = jnp.full((16,), 0x5F3759DF, jnp.int32) - lax.shift_right_logical(x, 1)
    y = plsc.bitcast(x, jnp.float32)
    for _ in range(3):
        y = y * (1.5 - (0.5 * d) * (y * y))
    return y


def _chunks(total, step):
    out = []
    off = 0
    while off < total:
        out.append((off, min(step, total - off)))
        off += step
    return out


@functools.partial(
    pl.kernel, mesh=_mesh, compiler_params=_sc_params,
    out_type=[jax.ShapeDtypeStruct((_E0,), jnp.float32),
              jax.ShapeDtypeStruct((_E1P,), jnp.float32)],
    scratch_types=[pltpu.VMEM((_N0P,), jnp.float32),
                   pltpu.VMEM((2048,), jnp.int32),
                   pltpu.VMEM((2048,), jnp.int32),
                   pltpu.VMEM((2048,), jnp.float32)],
)
def _norm_kernel(r0_hbm, c0_hbm, r1_hbm, c1_hbm, n0_hbm, n1_hbm,
                 dis, rb, cb, ob):
    wid = _wid()
    ones = jnp.full((16,), 1.0, jnp.float32)

    def level(rows_hbm, cols_hbm, out_hbm, n_nodes, n_pad, n_edges):
        def z(i, c):
            dis[pl.ds(i * 16, 16)] = _zeros16()
            return c
        lax.fori_loop(0, n_pad // 16, z, 0)
        for off, sz in _chunks(n_edges, 2048):
            pltpu.sync_copy(rows_hbm.at[pl.ds(off, sz)], rb.at[pl.ds(0, sz)])

            @plsc.parallel_loop(0, sz // 16, 1, unroll=4)
            def _(v):
                r16 = rb[pl.ds(v * 16, 16)]
                plsc.addupdate_scatter(dis, [r16], ones)
        def dz(i, c):
            d = dis[pl.ds(i * 16, 16)]
            y = _rsqrt16(jnp.maximum(d, 1.0))
            dis[pl.ds(i * 16, 16)] = jnp.where(d > 0.0, y, 0.0)
            return c
        lax.fori_loop(0, n_pad // 16, dz, 0)
        for off, sz in _chunks(n_edges, 2048):
            pltpu.sync_copy(rows_hbm.at[pl.ds(off, sz)], rb.at[pl.ds(0, sz)])
            pltpu.sync_copy(cols_hbm.at[pl.ds(off, sz)], cb.at[pl.ds(0, sz)])

            @plsc.parallel_loop(0, sz // 16, 1, unroll=4)
            def _(v):
                r16 = rb[pl.ds(v * 16, 16)]
                c16 = cb[pl.ds(v * 16, 16)]
                dr = plsc.load_gather(dis, [r16])
                dc = plsc.load_gather(dis, [c16])
                n = -(dr * dc)
                n = jnp.where(r16 < n_nodes, n, 0.0)
                ob[pl.ds(v * 16, 16)] = n
            pltpu.sync_copy(ob.at[pl.ds(0, sz)], out_hbm.at[pl.ds(off, sz)])

    @pl.when(wid == 0)
    def _():
        level(r0_hbm, c0_hbm, n0_hbm, _N0, _N0P, _E0)

    @pl.when(wid == 1)
    def _():
        level(r1_hbm, c1_hbm, n1_hbm, _N1, _N1P, _E1P)


_E0CH = 4096

@functools.partial(
    pl.kernel, mesh=_mesh, compiler_params=_sc_params,
    out_type=jax.ShapeDtypeStruct((_K - 1, _F0, _N0P), jnp.float32),
    scratch_types=[pltpu.VMEM((_N0P,), jnp.float32),
                   pltpu.VMEM((_N0P,), jnp.float32),
                   pltpu.VMEM((_N0P,), jnp.float32),
                   pltpu.VMEM((_N0P,), jnp.float32),
                   pltpu.VMEM((_E0CH,), jnp.int32),
                   pltpu.VMEM((_E0CH,), jnp.int32),
                   pltpu.VMEM((_E0CH,), jnp.float32),
                   pltpu.SemaphoreType.DMA],
)
def _cheb0_kernel(x_hbm, r_hbm, c_hbm, n_hbm, tcat_hbm,
                  a0, b0, a1, b1, rb, cb, nb, sem):
    wid = _wid()
    ch0 = wid
    ch1 = wid + 32
    has2 = wid < 16

    nvz = _N0P // 16

    pltpu.sync_copy(x_hbm.at[ch0], a0)

    @pl.when(has2)
    def _():
        pltpu.sync_copy(x_hbm.at[ch1], a1)

    def zb(i, c):
        b0[pl.ds(i * 16, 16)] = _zeros16()
        b1[pl.ds(i * 16, 16)] = _zeros16()
        return c
    lax.fori_loop(0, nvz, zb, 0)

    pairs = [(a0, b0), (a1, b1)]
    for k in range(1, _K):
        if k >= 2:
            def neg(i, c, d0=pairs[0][1], d1=pairs[1][1]):
                d0[pl.ds(i * 16, 16)] = -d0[pl.ds(i * 16, 16)]
                d1[pl.ds(i * 16, 16)] = -d1[pl.ds(i * 16, 16)]
                return c
            lax.fori_loop(0, nvz, neg, 0)
        for off, sz in _chunks(_E0, _E0CH):
            h1 = pltpu.async_copy(r_hbm.at[pl.ds(off, sz)], rb.at[pl.ds(0, sz)], sem)
            h2 = pltpu.async_copy(c_hbm.at[pl.ds(off, sz)], cb.at[pl.ds(0, sz)], sem)
            h3 = pltpu.async_copy(n_hbm.at[pl.ds(off, sz)], nb.at[pl.ds(0, sz)], sem)
            h1.wait(); h2.wait(); h3.wait()

            @plsc.parallel_loop(0, sz // 16, 1, unroll=4)
            def _(v):
                r16 = rb[pl.ds(v * 16, 16)]
                c16 = cb[pl.ds(v * 16, 16)]
                n16 = nb[pl.ds(v * 16, 16)]
                nn = n16 if k == 1 else n16 + n16
                s0, d0 = pairs[0]
                g = plsc.load_gather(s0, [c16])
                plsc.addupdate_scatter(d0, [r16], g * nn)

                @pl.when(has2)
                def _():
                    s1, d1 = pairs[1]
                    g1 = plsc.load_gather(s1, [c16])
                    plsc.addupdate_scatter(d1, [r16], g1 * nn)
        dst0 = pairs[0][1]
        pltpu.sync_copy(dst0, tcat_hbm.at[k - 1, ch0])

        @pl.when(has2)
        def _():
            dst1 = pairs[1][1]
            pltpu.sync_copy(dst1, tcat_hbm.at[k - 1, ch1])
        pairs = [(pairs[0][1], pairs[0][0]), (pairs[1][1], pairs[1][0])]


@functools.partial(
    pl.kernel, mesh=_mesh, compiler_params=_sc_params,
    out_type=jax.ShapeDtypeStruct((_K - 1, _F1, _N1P), jnp.float32),
    scratch_types=[pltpu.VMEM((16, _N1P), jnp.float32),
                   pltpu.VMEM((16, _N1P), jnp.float32),
                   pltpu.VMEM((_E1P,), jnp.int32),
                   pltpu.VMEM((_E1P,), jnp.int32),
                   pltpu.VMEM((_E1P,), jnp.float32),
                   pltpu.SemaphoreType.DMA],
)
def _cheb1_kernel(x_hbm, r_hbm, c_hbm, n_hbm, tcat_hbm,
                  abuf, bbuf, rb, cb, nb, sem):
    wid = _wid()
    h1 = pltpu.async_copy(r_hbm, rb, sem)
    h2 = pltpu.async_copy(c_hbm, cb, sem)
    h3 = pltpu.async_copy(n_hbm, nb, sem)
    h1.wait(); h2.wait(); h3.wait()

    nvz = _N1P // 16
    nv = _E1P // 16

    for p in range(2):
        ch0 = wid * 32 + p * 16
        pltpu.sync_copy(x_hbm.at[pl.ds(ch0, 16)], abuf)

        def zb(i, c):
            for j in range(16):
                bbuf[j, pl.ds(i * 16, 16)] = _zeros16()
            return c
        lax.fori_loop(0, nvz, zb, 0)

        src, dst = abuf, bbuf
        for k in range(1, _K):
            if k >= 2:
                def neg(i, c):
                    for j in range(16):
                        dst[j, pl.ds(i * 16, 16)] = -dst[j, pl.ds(i * 16, 16)]
                    return c
                lax.fori_loop(0, nvz, neg, 0)

            @plsc.parallel_loop(0, nv, 1, unroll=2)
            def _(v):
                r16 = rb[pl.ds(v * 16, 16)]
                c16 = cb[pl.ds(v * 16, 16)]
                n16 = nb[pl.ds(v * 16, 16)]
                nn = n16 if k == 1 else n16 + n16
                for j in range(16):
                    jj = jnp.full((16,), j, jnp.int32)
                    g = plsc.load_gather(src, [jj, c16])
                    plsc.addupdate_scatter(dst, [jj, r16], g * nn)
            pltpu.sync_copy(dst, tcat_hbm.at[k - 1, pl.ds(ch0, 16)])
            src, dst = dst, src


def _make_pool(F, NIN, NOUTP):
    per_tile = F // 32

    @functools.partial(
        pl.kernel, mesh=_mesh, compiler_params=_sc_params,
        out_type=jax.ShapeDtypeStruct((F, NOUTP), jnp.float32),
        scratch_types=[pltpu.VMEM((NIN,), jnp.float32),
                       pltpu.VMEM((4, NOUTP), jnp.int32),
                       pltpu.VMEM((4, NOUTP), jnp.float32),
                       pltpu.VMEM((NOUTP,), jnp.float32)],
    )
    def pool(h_hbm, cols_hbm, vals_hbm, out_hbm, xin, cbuf, vbuf, ob):
        wid = _wid()
        pltpu.sync_copy(cols_hbm, cbuf)
        pltpu.sync_copy(vals_hbm, vbuf)
        for ch in range(per_tile):
            c = wid * per_tile + ch
            pltpu.sync_copy(h_hbm.at[c], xin)

            @plsc.parallel_loop(0, NOUTP // 16, 1, unroll=4)
            def _(v):
                acc = _zeros16()
                for j in range(4):
                    idx = cbuf[j, pl.ds(v * 16, 16)]
                    g = plsc.load_gather(xin, [idx])
                    acc = acc + g * vbuf[j, pl.ds(v * 16, 16)]
                ob[pl.ds(v * 16, 16)] = acc
            pltpu.sync_copy(ob, out_hbm.at[c])

    return pool


_pool0 = _make_pool(_F1, _N0P, _N1P)
_pool1 = _make_pool(_F2, _N1P, _N2P)


def _elu(x):
    return jnp.where(x > 0.0, x, jnp.exp(jnp.minimum(x, 0.0)) - 1.0)


def _make_mm(CIN, COUT, NP):
    def body(x_ref, t_ref, w_ref, b_ref, o_ref):
        xb = x_ref[...].reshape(CIN, NP)
        tb = t_ref[...].reshape((_K - 1) * CIN, NP)
        lhs = jnp.concatenate([xb, tb], axis=0)
        o = lax.dot_general(w_ref[...], lhs, (((0,), (0,)), ((), ())),
                            preferred_element_type=jnp.float32)
        o_ref[...] = _elu(o + b_ref[...])

    return pl.pallas_call(
        body,
        grid=(_B,),
        in_specs=[
            pl.BlockSpec((1, CIN, NP), lambda b: (b, 0, 0)),
            pl.BlockSpec((_K - 1, 1, CIN, NP), lambda b: (0, b, 0, 0)),
            pl.BlockSpec((_K * CIN, COUT), lambda b: (0, 0)),
            pl.BlockSpec((COUT, 1), lambda b: (0, 0)),
        ],
        out_specs=pl.BlockSpec((COUT, NP), lambda b: (b, 0)),
        out_shape=jax.ShapeDtypeStruct((_B * COUT, NP), jnp.float32),
    )


_mm0 = _make_mm(_CIN, _C0, _N0P)
_mm1 = _make_mm(_C0, _C1, _N1P)


_KF = _C1 * _N2P
_KFB = 8192


def _mmf_body(l_ref, w_ref, b_ref, o_ref):
    i = pl.program_id(0)

    @pl.when(i == 0)
    def _():
        o_ref[...] = jnp.broadcast_to(b_ref[...], (_B, _LAT))

    o_ref[...] += lax.dot_general(
        l_ref[...], w_ref[...], (((1,), (0,)), ((), ())),
        preferred_element_type=jnp.float32)


_mmf = pl.pallas_call(
    _mmf_body,
    grid=(_KF // _KFB,),
    in_specs=[pl.BlockSpec((_B, _KFB), lambda i: (0, i)),
              pl.BlockSpec((_KFB, _LAT), lambda i: (i, 0)),
              pl.BlockSpec((1, _LAT), lambda i: (0, 0))],
    out_specs=pl.BlockSpec((_B, _LAT), lambda i: (0, 0)),
    out_shape=jax.ShapeDtypeStruct((_B, _LAT), jnp.float32),
)


def _deal(arr, lanes=16):
    r = arr.shape[0] // lanes
    return arr.reshape(lanes, r).T.reshape(-1)


def kernel(x, edge_index0, edge_index1, rows0, cols0, vals0, rows1, cols1,
           vals1, W0, b0, W1, b1, Wlin, blin):
    f32, i32 = jnp.float32, jnp.int32

    r0, c0 = edge_index0[0], edge_index0[1]
    p0 = jnp.argsort(r0)
    r0d = _deal(r0[p0]).astype(i32)
    c0d = _deal(c0[p0]).astype(i32)

    pad = _N1 + jnp.arange(_E1P - _E1, dtype=i32)
    r1 = jnp.concatenate([edge_index1[0].astype(i32), pad])
    c1 = jnp.concatenate([edge_index1[1].astype(i32), pad])
    p1 = jnp.argsort(r1)
    r1d = _deal(r1[p1])
    c1d = _deal(c1[p1])

    cols0r = jnp.pad(cols0.reshape(_N1, 4).T, ((0, 0), (0, _N1P - _N1))).astype(i32)
    vals0r = jnp.pad(vals0.reshape(_N1, 4).T, ((0, 0), (0, _N1P - _N1))).astype(f32)
    cols1r = jnp.pad(cols1.reshape(_N2, 4).T, ((0, 0), (0, _N2P - _N2))).astype(i32)
    vals1r = jnp.pad(vals1.reshape(_N2, 4).T, ((0, 0), (0, _N2P - _N2))).astype(f32)

    x0t = jnp.pad(x.transpose(0, 2, 1).reshape(_F0, _N0),
                  ((0, 0), (0, _N0P - _N0)))

    w0r = W0.reshape(_K * _CIN, _C0)
    w1r = W1.reshape(_K * _C0, _C1)
    b0c = b0.reshape(_C0, 1)
    b1c = b1.reshape(_C1, 1)
    wlin_t = jnp.pad(Wlin.reshape(_N2, _C1, _LAT).transpose(1, 0, 2),
                     ((0, 0), (0, _N2P - _N2), (0, 0))).reshape(_C1 * _N2P, _LAT)
    blin2 = blin.reshape(1, _LAT)

    n0, n1 = _norm_kernel(r0d, c0d, r1d, c1d)
    tcat0 = _cheb0_kernel(x0t, r0d, c0d, n0)
    h0 = _mm0(x0t.reshape(_B, _CIN, _N0P),
              tcat0.reshape(_K - 1, _B, _CIN, _N0P), w0r, b0c)
    x1 = _pool0(h0, cols0r, vals0r)
    tcat1 = _cheb1_kernel(x1, r1d, c1d, n1)
    h1 = _mm1(x1.reshape(_B, _C0, _N1P),
              tcat1.reshape(_K - 1, _B, _C0, _N1P), w1r, b1c)
    h1p = _pool1(h1, cols1r, vals1r)
    lhs = h1p.reshape(_B, _C1 * _N2P)
    return _mmf(lhs, wlin_t, blin2)
